# baseline (speedup 1.0000x reference)
import functools

import jax
import jax.numpy as jnp
from jax.experimental import pallas as pl
from jax.experimental.pallas import tpu as pltpu


def _encoder_kernel(flat_ref, pos_ref, time_ref, pol_ref, out_ref,
                    bind_ref, acc_ref, *, nb):
    n = pl.program_id(0)

    @pl.when(n == 0)
    def _init():
        acc_ref[...] = jnp.zeros_like(acc_ref)
        tc, bd = bind_ref.shape
        row = jax.lax.broadcasted_iota(jnp.int32, (tc, bd), 0)
        pol = pol_ref[...]
        pol_alt = jnp.where((row & 1) == 0, pol[0:1, :], pol[1:2, :])
        r_row = jax.lax.broadcasted_iota(jnp.int32, (tc, tc // 2), 0)
        r_col = jax.lax.broadcasted_iota(jnp.int32, (tc, tc // 2), 1)
        rep = (r_col == (r_row // 2)).astype(jnp.float32)
        time_rep = jnp.dot(rep, time_ref[...],
                           preferred_element_type=jnp.float32)
        bind_ref[...] = pol_alt * time_rep

    tc = flat_ref.shape[0]
    bn = pos_ref.shape[0]
    bh = bn // flat_ref.shape[2]
    flat_blk = flat_ref[:, pl.dslice(n * bh, bh), :].reshape(tc, bn)
    b_blk = jax.lax.dot_general(
        flat_blk, bind_ref[...],
        dimension_numbers=(((0,), (0,)), ((), ())),
        preferred_element_type=jnp.float32)
    acc_ref[...] += jnp.sum(pos_ref[...] * b_blk, axis=0)[None, :]

    @pl.when(n == nb - 1)
    def _finish():
        out_ref[...] = jnp.sign(acc_ref[...])


def kernel(hist_tensor, time_weight, pol_weight, pos_weight):
    T, C, H, W = hist_tensor.shape
    HW = H * W
    max_time, D = time_weight.shape
    if T <= max_time:
        time_hv = time_weight[:T]
    else:
        time_hv = jnp.take(time_weight,
                           jnp.minimum(jnp.arange(T), max_time - 1), axis=0)
    flat = hist_tensor.reshape(T * C, H, W)

    BN = 512
    BH = BN // W
    nb = HW // BN

    out = pl.pallas_call(
        functools.partial(_encoder_kernel, nb=nb),
        grid=(nb,),
        in_specs=[
            pl.BlockSpec((T * C, H, W), lambda n: (0, 0, 0)),
            pl.BlockSpec((BN, D), lambda n: (n, 0)),
            pl.BlockSpec((T, D), lambda n: (0, 0)),
            pl.BlockSpec((2, D), lambda n: (0, 0)),
        ],
        out_specs=pl.BlockSpec((1, D), lambda n: (0, 0)),
        out_shape=jax.ShapeDtypeStruct((1, D), jnp.float32),
        scratch_shapes=[pltpu.VMEM((T * C, D), jnp.float32),
                        pltpu.VMEM((1, D), jnp.float32)],
        compiler_params=pltpu.CompilerParams(
            dimension_semantics=("arbitrary",),
            vmem_limit_bytes=100 * 1024 * 1024,
        ),
    )(flat, pos_weight, time_hv, pol_weight)
    return out.reshape(D)


# speedup vs baseline: 1.0096x; 1.0096x over previous
import functools

import jax
import jax.numpy as jnp
from jax.experimental import pallas as pl
from jax.experimental.pallas import tpu as pltpu


def _encoder_kernel(flat_ref, pos_ref, time_ref, pol_ref, out_ref,
                    bind_ref, acc_ref, *, nb):
    n = pl.program_id(0)
    j = pl.program_id(1)

    @pl.when((n == 0) & (j == 0))
    def _init():
        acc_ref[...] = jnp.zeros_like(acc_ref)
        tc, bd = bind_ref.shape
        row = jax.lax.broadcasted_iota(jnp.int32, (tc, bd), 0)
        pol = pol_ref[...]
        pol_alt = jnp.where((row & 1) == 0, pol[0:1, :], pol[1:2, :])
        r_row = jax.lax.broadcasted_iota(jnp.int32, (tc, tc // 2), 0)
        r_col = jax.lax.broadcasted_iota(jnp.int32, (tc, tc // 2), 1)
        rep = (r_col == (r_row // 2)).astype(jnp.float32)
        time_rep = jnp.dot(rep, time_ref[...],
                           preferred_element_type=jnp.float32)
        bind_ref[...] = pol_alt * time_rep

    tc = flat_ref.shape[0]
    bn = flat_ref.shape[1] * flat_ref.shape[2]
    hd = pos_ref.shape[1]
    flat_blk = flat_ref[...].reshape(tc, bn)
    b_blk = jax.lax.dot_general(
        flat_blk, bind_ref[:, pl.dslice(j * hd, hd)],
        dimension_numbers=(((0,), (0,)), ((), ())),
        preferred_element_type=jnp.float32)
    acc_ref[:, pl.dslice(j * hd, hd)] += jnp.sum(
        pos_ref[...] * b_blk, axis=0)[None, :]

    @pl.when((n == nb - 1) & (j == 1))
    def _finish():
        out_ref[...] = jnp.sign(acc_ref[...])


def kernel(hist_tensor, time_weight, pol_weight, pos_weight):
    T, C, H, W = hist_tensor.shape
    HW = H * W
    max_time, D = time_weight.shape
    if T <= max_time:
        time_hv = time_weight[:T]
    else:
        time_hv = jnp.take(time_weight,
                           jnp.minimum(jnp.arange(T), max_time - 1), axis=0)
    flat = hist_tensor.reshape(T * C, H, W)

    BN = 1024
    BH = BN // W
    nb = HW // BN

    out = pl.pallas_call(
        functools.partial(_encoder_kernel, nb=nb),
        grid=(nb, 2),
        in_specs=[
            pl.BlockSpec((T * C, BH, W), lambda n, j: (0, n, 0)),
            pl.BlockSpec((BN, D // 2), lambda n, j: (n, j)),
            pl.BlockSpec((T, D), lambda n, j: (0, 0)),
            pl.BlockSpec((2, D), lambda n, j: (0, 0)),
        ],
        out_specs=pl.BlockSpec((1, D), lambda n, j: (0, 0)),
        out_shape=jax.ShapeDtypeStruct((1, D), jnp.float32),
        scratch_shapes=[pltpu.VMEM((T * C, D), jnp.float32),
                        pltpu.VMEM((1, D), jnp.float32)],
        compiler_params=pltpu.CompilerParams(
            dimension_semantics=("arbitrary", "arbitrary"),
            vmem_limit_bytes=100 * 1024 * 1024,
        ),
    )(flat, pos_weight, time_hv, pol_weight)
    return out.reshape(D)


# speedup vs baseline: 1.2452x; 1.2333x over previous
import functools

import jax
import jax.numpy as jnp
from jax.experimental import pallas as pl
from jax.experimental.pallas import tpu as pltpu


def _encoder_kernel(flat_ref, pos_ref, time_ref, pol_ref, out_ref,
                    bind_ref, acc_ref, *, nb):
    n = pl.program_id(0)

    @pl.when(n == 0)
    def _init():
        acc_ref[...] = jnp.zeros_like(acc_ref)
        tc, bd = bind_ref.shape
        row = jax.lax.broadcasted_iota(jnp.int32, (tc, bd), 0)
        pol = pol_ref[...]
        pol_alt = jnp.where((row & 1) == 0, pol[0:1, :], pol[1:2, :])
        r_row = jax.lax.broadcasted_iota(jnp.int32, (tc, tc // 2), 0)
        r_col = jax.lax.broadcasted_iota(jnp.int32, (tc, tc // 2), 1)
        rep = (r_col == (r_row // 2)).astype(jnp.float32)
        time_rep = jnp.dot(rep, time_ref[...],
                           preferred_element_type=jnp.float32)
        bind_ref[...] = pol_alt * time_rep

    tc = flat_ref.shape[0]
    bn = flat_ref.shape[1] * flat_ref.shape[2]
    flat_blk = flat_ref[...].reshape(tc, bn)
    b_blk = jax.lax.dot_general(
        flat_blk, bind_ref[...],
        dimension_numbers=(((0,), (0,)), ((), ())),
        preferred_element_type=jnp.float32)
    acc_ref[...] += jnp.sum(pos_ref[...] * b_blk, axis=0)[None, :]

    @pl.when(n == nb - 1)
    def _finish():
        out_ref[...] = jnp.sign(acc_ref[...])


def kernel(hist_tensor, time_weight, pol_weight, pos_weight):
    T, C, H, W = hist_tensor.shape
    HW = H * W
    max_time, D = time_weight.shape
    if T <= max_time:
        time_hv = time_weight[:T]
    else:
        time_hv = jnp.take(time_weight,
                           jnp.minimum(jnp.arange(T), max_time - 1), axis=0)
    flat = hist_tensor.reshape(T * C, H, W)

    BN = 1024
    BH = BN // W
    nb = HW // BN

    out = pl.pallas_call(
        functools.partial(_encoder_kernel, nb=nb),
        grid=(nb,),
        in_specs=[
            pl.BlockSpec((T * C, BH, W), lambda n: (0, n, 0)),
            pl.BlockSpec((BN, D), lambda n: (n, 0)),
            pl.BlockSpec((T, D), lambda n: (0, 0)),
            pl.BlockSpec((2, D), lambda n: (0, 0)),
        ],
        out_specs=pl.BlockSpec((1, D), lambda n: (0, 0)),
        out_shape=jax.ShapeDtypeStruct((1, D), jnp.float32),
        scratch_shapes=[pltpu.VMEM((T * C, D), jnp.float32),
                        pltpu.VMEM((1, D), jnp.float32)],
        compiler_params=pltpu.CompilerParams(
            dimension_semantics=("arbitrary",),
            vmem_limit_bytes=100 * 1024 * 1024,
        ),
    )(flat, pos_weight, time_hv, pol_weight)
    return out.reshape(D)
